# baseline (speedup 1.0000x reference)
import functools
import jax
import jax.numpy as jnp
from jax import lax
from jax.experimental import pallas as pl
from jax.experimental.pallas import tpu as pltpu
from jax.experimental.pallas import tpu_sc as plsc

B, C, HW, P, OUT = 16384, 11, 64, 32, 64
CP = 16
NW = 32
BPT = B // NW
GRP = 16
NG = BPT // GRP
IDS_G = GRP * HW
BRD_G = GRP * C * HW
RAW_G = GRP * P * CP


def _sc_kernel_body(ids_hbm, board_hbm, raw_hbm, ids_v, board_v, table_v, raw_v):
    wid = lax.axis_index("s") * 2 + lax.axis_index("c")
    lanes = lax.iota(jnp.int32, 16)
    lane64 = lanes * HW
    lane33 = lanes * 33
    lane704 = lanes * (C * HW)
    lane512 = lanes * (P * CP)
    zeros_f = jnp.zeros((16,), jnp.float32)
    sent = jnp.full((16,), HW, jnp.int32)

    def zinit(i, _):
        plsc.store_scatter(raw_v, [lanes + i * 16], zeros_f)
        return _
    lax.fori_loop(0, RAW_G // 16, zinit, None)

    def group(g, _):
        base = wid * BPT + g * GRP
        pltpu.sync_copy(ids_hbm.at[pl.ds(base * HW, IDS_G)], ids_v)
        pltpu.sync_copy(board_hbm.at[pl.ds(base * C * HW, BRD_G)], board_v)

        def tinit(i, _):
            plsc.store_scatter(table_v, [lanes + i * 16], sent)
            return _
        lax.fori_loop(0, 33, tinit, None)

        def scan_cell(j, _):
            jj = 63 - j
            ids_vec = plsc.load_gather(ids_v, [lane64 + jj])
            plsc.store_scatter(table_v, [lane33 + ids_vec],
                               jnp.full((16,), jj, jnp.int32))
            return _
        lax.fori_loop(0, HW, scan_cell, None)

        def piece(p, _):
            fvec = plsc.load_gather(table_v, [lane33 + (p + 1)])
            msk = fvec < HW
            addr = lane704 + jnp.where(msk, fvec, 0)
            for c in range(C):
                val = plsc.load_gather(board_v, [addr + c * HW])
                val = jnp.where(msk, val, 0.0)
                plsc.store_scatter(raw_v, [lane512 + (p * CP + c)], val)
            return _
        lax.fori_loop(0, P, piece, None)

        pltpu.sync_copy(raw_v, raw_hbm.at[pl.ds(base * P * CP, RAW_G)])
        return _
    lax.fori_loop(0, NG, group, None)


def _sc_extract(ids_flat, board_flat):
    mesh = plsc.VectorSubcoreMesh(core_axis_name="c", subcore_axis_name="s")
    k = functools.partial(
        pl.kernel,
        mesh=mesh,
        compiler_params=pltpu.CompilerParams(needs_layout_passes=False),
        out_type=jax.ShapeDtypeStruct((B * P * CP,), jnp.float32),
        scratch_types=[
            pltpu.VMEM((IDS_G,), jnp.int32),
            pltpu.VMEM((BRD_G,), jnp.float32),
            pltpu.VMEM((33 * 16,), jnp.int32),
            pltpu.VMEM((RAW_G,), jnp.float32),
        ],
    )(_sc_kernel_body)
    return k(ids_flat, board_flat)


def _tc_body(raw_ref, wt_ref, bias_ref, out_ref):
    out_ref[...] = jnp.dot(raw_ref[...], wt_ref[...],
                           preferred_element_type=jnp.float32) + bias_ref[...]


MBLK = 4096


@jax.jit
def kernel(full_board_vector, piece_ids, proj_w, proj_b):
    ids_flat = piece_ids.reshape(B * HW)
    board_flat = full_board_vector.reshape(B * C * HW)
    raw = _sc_extract(ids_flat, board_flat).reshape(B * P, CP)

    wt = jnp.zeros((CP, OUT), jnp.float32).at[:C, :].set(proj_w.T)
    bias = proj_b.reshape(1, OUT)

    out = pl.pallas_call(
        _tc_body,
        grid=(B * P // MBLK,),
        in_specs=[
            pl.BlockSpec((MBLK, CP), lambda i: (i, 0)),
            pl.BlockSpec((CP, OUT), lambda i: (0, 0)),
            pl.BlockSpec((1, OUT), lambda i: (0, 0)),
        ],
        out_specs=pl.BlockSpec((MBLK, OUT), lambda i: (i, 0)),
        out_shape=jax.ShapeDtypeStruct((B * P, OUT), jnp.float32),
    )(raw, wt, bias)
    return out.reshape(B, P, OUT)


# speedup vs baseline: 3.8336x; 3.8336x over previous
import functools
import jax
import jax.numpy as jnp
from jax import lax
from jax.experimental import pallas as pl
from jax.experimental.pallas import tpu as pltpu
from jax.experimental.pallas import tpu_sc as plsc

B, C, HW, P, OUT = 16384, 11, 64, 32, 64
NW = 32
BPT = B // NW
BC = 128
NCHUNK = BPT // BC
NLG = BC // 16
PB = 8


def _sc_kernel_body(ids_hbm, board_hbm, raw_hbm, ids_v, board_v, table_v, raw_v):
    wid = lax.axis_index("s") * 2 + lax.axis_index("c")
    lanes = lax.iota(jnp.int32, 16)
    sent = jnp.full((16,), HW, jnp.int32)

    def chunk(g, _):
        cb = pl.multiple_of(wid * BPT + g * BC, BC)
        pltpu.sync_copy(ids_hbm.at[:, pl.ds(cb, BC)], ids_v)
        pltpu.sync_copy(board_hbm.at[:, pl.ds(cb, BC)], board_v)

        for t in range(33):
            for lg in range(NLG):
                table_v[t, pl.ds(lg * 16, 16)] = sent

        for lg in range(NLG):
            col = lanes + lg * 16

            def scan_cell(j, _):
                jj = 63 - j
                jjv = jnp.full((16,), jj, jnp.int32)
                ids_vec = plsc.load_gather(ids_v, [jjv, col])
                plsc.store_scatter(table_v, [ids_vec, col], jjv)
                return _
            lax.fori_loop(0, HW, scan_cell, None)

        for pb in range(P // PB):
            def piece(p, _):
                tv = jnp.full((16,), pb * PB + 1, jnp.int32) + p
                pv = jnp.zeros((16,), jnp.int32) + p
                for lg in range(NLG):
                    col = lanes + lg * 16
                    fvec = plsc.load_gather(table_v, [tv, col])
                    msk = fvec < HW
                    f0 = jnp.where(msk, fvec, 0)
                    for c in range(C):
                        val = plsc.load_gather(board_v, [f0 + c * HW, col])
                        val = jnp.where(msk, val, 0.0)
                        plsc.store_scatter(raw_v, [pv * C + c, col], val)
                return _
            lax.fori_loop(0, PB, piece, None)
            pltpu.sync_copy(raw_v,
                            raw_hbm.at[pl.ds(pb * PB * C, PB * C),
                                       pl.ds(cb, BC)])
        return _
    lax.fori_loop(0, NCHUNK, chunk, None)


def _sc_extract(ids_t, board_t):
    mesh = plsc.VectorSubcoreMesh(core_axis_name="c", subcore_axis_name="s")
    k = functools.partial(
        pl.kernel,
        mesh=mesh,
        compiler_params=pltpu.CompilerParams(needs_layout_passes=False),
        out_type=jax.ShapeDtypeStruct((P * C, B), jnp.float32),
        scratch_types=[
            pltpu.VMEM((HW, BC), jnp.int32),
            pltpu.VMEM((C * HW, BC), jnp.float32),
            pltpu.VMEM((33, BC), jnp.int32),
            pltpu.VMEM((PB * C, BC), jnp.float32),
        ],
    )(_sc_kernel_body)
    return k(ids_t, board_t)


def _tc_body(raw_ref, w_ref, bias_ref, out_ref):
    out_ref[0] = jnp.dot(w_ref[...], raw_ref[0],
                         preferred_element_type=jnp.float32) + bias_ref[...]


BN = 2048


@jax.jit
def kernel(full_board_vector, piece_ids, proj_w, proj_b):
    ids_t = piece_ids.transpose(1, 2, 0).reshape(HW, B)
    board_t = full_board_vector.transpose(1, 2, 3, 0).reshape(C * HW, B)

    raw = _sc_extract(ids_t, board_t).reshape(P, C, B)
    bias = proj_b.reshape(OUT, 1)

    out = pl.pallas_call(
        _tc_body,
        grid=(P, B // BN),
        in_specs=[
            pl.BlockSpec((1, C, BN), lambda p, i: (p, 0, i)),
            pl.BlockSpec((OUT, C), lambda p, i: (0, 0)),
            pl.BlockSpec((OUT, 1), lambda p, i: (0, 0)),
        ],
        out_specs=pl.BlockSpec((1, OUT, BN), lambda p, i: (p, 0, i)),
        out_shape=jax.ShapeDtypeStruct((P, OUT, B), jnp.float32),
    )(raw, proj_w, bias)
    return out.transpose(2, 0, 1)


# speedup vs baseline: 5.4722x; 1.4274x over previous
import functools
import jax
import jax.numpy as jnp
from jax import lax
from jax.experimental import pallas as pl
from jax.experimental.pallas import tpu as pltpu
from jax.experimental.pallas import tpu_sc as plsc

B, C, HW, P, OUT = 16384, 11, 64, 32, 64
NW = 32
BPT = B // NW
BC = 128
NCHUNK = BPT // BC
NLG = BC // 16
PB = 8


def _sc_kernel_body(ids_hbm, board_hbm, raw_hbm, ids_v, board_v, table_v, raw_v):
    wid = lax.axis_index("s") * 2 + lax.axis_index("c")
    lanes = lax.iota(jnp.int32, 16)
    sent = jnp.full((16,), HW, jnp.int32)

    def chunk(g, _):
        cb = pl.multiple_of(wid * BPT + g * BC, BC)
        pltpu.sync_copy(ids_hbm.at[:, pl.ds(cb, BC)], ids_v)
        pltpu.sync_copy(board_hbm.at[:, pl.ds(cb, BC)], board_v)

        for t in range(33):
            for lg in range(NLG):
                table_v[t, pl.ds(lg * 16, 16)] = sent

        for lg in range(NLG):
            col = lanes + lg * 16

            def scan_cell(j, _):
                jj = 63 - j
                jjv = jnp.full((16,), jj, jnp.int32)
                ids_vec = plsc.load_gather(ids_v, [jjv, col])
                plsc.store_scatter(table_v, [ids_vec, col], jjv)
                return _
            lax.fori_loop(0, HW, scan_cell, None)

        for pb in range(P // PB):
            def piece(p, _):
                tv = jnp.full((16,), pb * PB + 1, jnp.int32) + p
                pv = jnp.zeros((16,), jnp.int32) + p
                for lg in range(NLG):
                    col = lanes + lg * 16
                    fvec = plsc.load_gather(table_v, [tv, col])
                    msk = fvec < HW
                    f0 = jnp.where(msk, fvec, 0)
                    for c in range(C):
                        val = plsc.load_gather(board_v, [f0 + c * HW, col])
                        val = jnp.where(msk, val, 0.0)
                        plsc.store_scatter(raw_v, [pv * C + c, col], val)
                return _
            lax.fori_loop(0, PB, piece, None)
            pltpu.sync_copy(raw_v,
                            raw_hbm.at[pl.ds(pb * PB * C, PB * C),
                                       pl.ds(cb, BC)])
        return _
    lax.fori_loop(0, NCHUNK, chunk, None)


def _sc_extract(ids_t, board_t):
    mesh = plsc.VectorSubcoreMesh(core_axis_name="c", subcore_axis_name="s")
    k = functools.partial(
        pl.kernel,
        mesh=mesh,
        compiler_params=pltpu.CompilerParams(needs_layout_passes=False),
        out_type=jax.ShapeDtypeStruct((P * C, B), jnp.float32),
        scratch_types=[
            pltpu.VMEM((HW, BC), jnp.int32),
            pltpu.VMEM((C * HW, BC), jnp.float32),
            pltpu.VMEM((33, BC), jnp.int32),
            pltpu.VMEM((PB * C, BC), jnp.float32),
        ],
    )(_sc_kernel_body)
    return k(ids_t, board_t)


def _tc_body(raw_ref, w_ref, bias_ref, out_ref):
    out_ref[0] = jnp.dot(w_ref[...], raw_ref[0],
                         preferred_element_type=jnp.float32) + bias_ref[...]


BN = 8192


@jax.jit
def kernel(full_board_vector, piece_ids, proj_w, proj_b):
    ids_t = piece_ids.transpose(1, 2, 0).reshape(HW, B)
    board_t = full_board_vector.transpose(1, 2, 3, 0).reshape(C * HW, B)

    raw = _sc_extract(ids_t, board_t).reshape(P, C, B)
    bias = proj_b.reshape(OUT, 1)

    out = pl.pallas_call(
        _tc_body,
        grid=(P, B // BN),
        in_specs=[
            pl.BlockSpec((1, C, BN), lambda p, i: (p, 0, i)),
            pl.BlockSpec((OUT, C), lambda p, i: (0, 0)),
            pl.BlockSpec((OUT, 1), lambda p, i: (0, 0)),
        ],
        out_specs=pl.BlockSpec((1, OUT, BN), lambda p, i: (p, 0, i)),
        out_shape=jax.ShapeDtypeStruct((P, OUT, B), jnp.float32),
    )(raw, proj_w, bias)
    return out.transpose(2, 0, 1)


# speedup vs baseline: 5.9272x; 1.0831x over previous
import functools
import jax
import jax.numpy as jnp
from jax import lax
from jax.experimental import pallas as pl
from jax.experimental.pallas import tpu as pltpu
from jax.experimental.pallas import tpu_sc as plsc

B, C, HW, P, OUT = 16384, 11, 64, 32, 64
NW = 32
BPT = B // NW
BC = 128
NCHUNK = BPT // BC
NLG = BC // 16
PB = 8


def _sc_kernel_body(ids_hbm, board_hbm, raw_hbm, ids_v, board_v, table_v, raw_v):
    wid = lax.axis_index("s") * 2 + lax.axis_index("c")
    lanes = lax.iota(jnp.int32, 16)
    sent = jnp.full((16,), HW, jnp.int32)

    def chunk(g, _):
        cb = pl.multiple_of(wid * BPT + g * BC, BC)
        pltpu.sync_copy(ids_hbm.at[:, pl.ds(cb, BC)], ids_v)
        pltpu.sync_copy(board_hbm.at[:, pl.ds(cb, BC)], board_v)

        for t in range(33):
            for lg in range(NLG):
                table_v[t, pl.ds(lg * 16, 16)] = sent

        for lg in range(NLG):
            col = lanes + lg * 16

            def scan_cell(j, _):
                jj = 63 - j
                jjv = jnp.full((16,), jj, jnp.int32)
                ids_vec = plsc.load_gather(ids_v, [jjv, col])
                plsc.store_scatter(table_v, [ids_vec, col], jjv)
                return _
            lax.fori_loop(0, HW, scan_cell, None)

        for pb in range(P // PB):
            def piece(p, _):
                tv = jnp.full((16,), pb * PB + 1, jnp.int32) + p
                pv = jnp.zeros((16,), jnp.int32) + p
                for lg in range(NLG):
                    col = lanes + lg * 16
                    fvec = plsc.load_gather(table_v, [tv, col])
                    msk = fvec < HW
                    f0 = jnp.where(msk, fvec, 0)
                    for c in range(C):
                        val = plsc.load_gather(board_v, [f0 + c * HW, col])
                        val = jnp.where(msk, val, 0.0)
                        plsc.store_scatter(raw_v, [pv * C + c, col], val)
                return _
            lax.fori_loop(0, PB, piece, None)
            pltpu.sync_copy(raw_v,
                            raw_hbm.at[pl.ds(pb * PB * C, PB * C),
                                       pl.ds(cb, BC)])
        return _
    lax.fori_loop(0, NCHUNK, chunk, None)


def _sc_extract(ids_t, board_t):
    mesh = plsc.VectorSubcoreMesh(core_axis_name="c", subcore_axis_name="s")
    k = functools.partial(
        pl.kernel,
        mesh=mesh,
        compiler_params=pltpu.CompilerParams(needs_layout_passes=False),
        out_type=jax.ShapeDtypeStruct((P * C, B), jnp.float32),
        scratch_types=[
            pltpu.VMEM((HW, BC), jnp.int32),
            pltpu.VMEM((C * HW, BC), jnp.float32),
            pltpu.VMEM((33, BC), jnp.int32),
            pltpu.VMEM((PB * C, BC), jnp.float32),
        ],
    )(_sc_kernel_body)
    return k(ids_t, board_t)


def _tc_body(raw_ref, w_ref, bias_ref, out_ref):
    out_ref[0] = jnp.dot(w_ref[...], raw_ref[0],
                         preferred_element_type=jnp.float32) + bias_ref[...]


BN = 16384


@jax.jit
def kernel(full_board_vector, piece_ids, proj_w, proj_b):
    ids_t = piece_ids.transpose(1, 2, 0).reshape(HW, B)
    board_t = full_board_vector.transpose(1, 2, 3, 0).reshape(C * HW, B)

    raw = _sc_extract(ids_t, board_t).reshape(P, C, B)
    bias = proj_b.reshape(OUT, 1)

    out = pl.pallas_call(
        _tc_body,
        grid=(P, B // BN),
        in_specs=[
            pl.BlockSpec((1, C, BN), lambda p, i: (p, 0, i)),
            pl.BlockSpec((OUT, C), lambda p, i: (0, 0)),
            pl.BlockSpec((OUT, 1), lambda p, i: (0, 0)),
        ],
        out_specs=pl.BlockSpec((1, OUT, BN), lambda p, i: (p, 0, i)),
        out_shape=jax.ShapeDtypeStruct((P, OUT, B), jnp.float32),
    )(raw, proj_w, bias)
    return out.transpose(2, 0, 1)


# speedup vs baseline: 7.3039x; 1.2323x over previous
import functools
import jax
import jax.numpy as jnp
from jax import lax
from jax.experimental import pallas as pl
from jax.experimental.pallas import tpu as pltpu
from jax.experimental.pallas import tpu_sc as plsc

B, C, HW, P, OUT = 16384, 11, 64, 32, 64
NW = 32
BPT = B // NW
BC = 128
NCHUNK = BPT // BC
NLG = BC // 16
PB = 8


def _sc_kernel_body(ids_hbm, board_hbm, raw_hbm, ids_v, board_v, table_v,
                    raw_a, raw_b, sem_ids, sem_board, sem_out):
    wid = lax.axis_index("s") * 2 + lax.axis_index("c")
    lanes = lax.iota(jnp.int32, 16)
    sent = jnp.full((16,), HW, jnp.int32)

    def chunk(g, _):
        cb = pl.multiple_of(wid * BPT + g * BC, BC)
        cp_ids = pltpu.async_copy(ids_hbm.at[:, pl.ds(cb, BC)], ids_v,
                                  sem_ids)
        cp_board = pltpu.async_copy(board_hbm.at[:, pl.ds(cb, BC)], board_v,
                                    sem_board)

        for t in range(33):
            for lg in range(NLG):
                table_v[t, pl.ds(lg * 16, 16)] = sent
        cp_ids.wait()

        def scan_cell(j, _):
            jj = 63 - j
            jjv = jnp.full((16,), jj, jnp.int32)
            for lg in range(NLG):
                col = lanes + lg * 16
                ids_vec = plsc.load_gather(ids_v, [jjv, col])
                plsc.store_scatter(table_v, [ids_vec, col], jjv)
            return _
        lax.fori_loop(0, HW, scan_cell, None)
        cp_board.wait()

        out_cps = {}
        for pb in range(P // PB):
            buf = raw_a if pb % 2 == 0 else raw_b
            if pb >= 2:
                out_cps[pb - 2].wait()

            @plsc.parallel_loop(0, PB, 1, unroll=2)
            def piece(p):
                tv = jnp.full((16,), pb * PB + 1, jnp.int32) + p
                pv = jnp.zeros((16,), jnp.int32) + p
                for lg in range(NLG):
                    col = lanes + lg * 16
                    fvec = plsc.load_gather(table_v, [tv, col])
                    msk = fvec < HW
                    f0 = jnp.where(msk, fvec, 0)
                    for c in range(C):
                        val = plsc.load_gather(board_v, [f0 + c * HW, col])
                        val = jnp.where(msk, val, 0.0)
                        plsc.store_scatter(buf, [pv * C + c, col], val)

            out_cps[pb] = pltpu.async_copy(
                buf, raw_hbm.at[pl.ds(pb * PB * C, PB * C), pl.ds(cb, BC)],
                sem_out)
        out_cps[2].wait()
        out_cps[3].wait()
        return _
    lax.fori_loop(0, NCHUNK, chunk, None)


def _sc_extract(ids_t, board_t):
    mesh = plsc.VectorSubcoreMesh(core_axis_name="c", subcore_axis_name="s")
    k = functools.partial(
        pl.kernel,
        mesh=mesh,
        compiler_params=pltpu.CompilerParams(needs_layout_passes=False),
        out_type=jax.ShapeDtypeStruct((P * C, B), jnp.float32),
        scratch_types=[
            pltpu.VMEM((HW, BC), jnp.int32),
            pltpu.VMEM((C * HW, BC), jnp.float32),
            pltpu.VMEM((33, BC), jnp.int32),
            pltpu.VMEM((PB * C, BC), jnp.float32),
            pltpu.VMEM((PB * C, BC), jnp.float32),
            pltpu.SemaphoreType.DMA,
            pltpu.SemaphoreType.DMA,
            pltpu.SemaphoreType.DMA,
        ],
    )(_sc_kernel_body)
    return k(ids_t, board_t)


def _tc_body(raw_ref, w_ref, bias_ref, out_ref):
    out_ref[0] = jnp.dot(w_ref[...], raw_ref[0],
                         preferred_element_type=jnp.float32) + bias_ref[...]


BN = 16384


@jax.jit
def kernel(full_board_vector, piece_ids, proj_w, proj_b):
    ids_t = piece_ids.transpose(1, 2, 0).reshape(HW, B)
    board_t = full_board_vector.transpose(1, 2, 3, 0).reshape(C * HW, B)

    raw = _sc_extract(ids_t, board_t).reshape(P, C, B)
    bias = proj_b.reshape(OUT, 1)

    out = pl.pallas_call(
        _tc_body,
        grid=(P, B // BN),
        in_specs=[
            pl.BlockSpec((1, C, BN), lambda p, i: (p, 0, i)),
            pl.BlockSpec((OUT, C), lambda p, i: (0, 0)),
            pl.BlockSpec((OUT, 1), lambda p, i: (0, 0)),
        ],
        out_specs=pl.BlockSpec((1, OUT, BN), lambda p, i: (p, 0, i)),
        out_shape=jax.ShapeDtypeStruct((P, OUT, B), jnp.float32),
    )(raw, proj_w, bias)
    return out.transpose(2, 0, 1)
